# Initial kernel scaffold; baseline (speedup 1.0000x reference)
import jax
import jax.numpy as jnp
from jax.experimental import pallas as pl


def kernel(nodes, neigh_idx, features, W_enc, weight):
    raise NotImplementedError("write your pallas kernel here")


# speedup vs baseline: 7.1702x; 7.1702x over previous
import jax
import jax.numpy as jnp
from jax import lax
from jax.experimental import pallas as pl
from jax.experimental.pallas import tpu as pltpu
from jax.experimental.pallas import tpu_sc as plsc

B = 16384
D = 128
S = 25
C = 64
NC = 2
NS = 16
NW = NC * NS
PER_W = B // NW
K = 16
CHUNKS = PER_W // K
GLEN = 100
GPC = (K * S) // GLEN
NLANE = 16
NVD = D // NLANE


def _sc_body(nodes_hbm, neigh_hbm, feat_hbm, self_out, sum_out,
             idxs_v, idxn_v, rows_v, self_v, sum_v, sem):
    cid = lax.axis_index("c")
    sid = lax.axis_index("s")
    wid = sid * NC + cid
    base0 = wid * PER_W

    @pl.loop(0, CHUNKS)
    def _chunk(c):
        t = wid * CHUNKS + c
        base = t * K
        pltpu.sync_copy(nodes_hbm.at[pl.ds(base, K)], idxs_v)
        pltpu.sync_copy(neigh_hbm.at[pl.ds(t * GPC, GPC)], idxn_v)
        cps = []
        for j in range(GPC):
            cps.append(pltpu.async_copy(
                feat_hbm.at[idxn_v.at[j]],
                rows_v.at[pl.ds(j * GLEN, GLEN)], sem))
        cps.append(pltpu.async_copy(feat_hbm.at[idxs_v], self_v, sem))
        for cp in cps:
            cp.wait()

        @pl.loop(0, K)
        def _elem(k):
            r0 = k * S
            acc = tuple(rows_v[r0, pl.ds(NLANE * d, NLANE)]
                        for d in range(NVD))

            def _sbody(s2, a):
                return tuple(a[d] + rows_v[r0 + s2, pl.ds(NLANE * d, NLANE)]
                             for d in range(NVD))

            acc = lax.fori_loop(1, S, _sbody, acc)
            for d in range(NVD):
                sum_v[k, pl.ds(NLANE * d, NLANE)] = acc[d]

        pltpu.sync_copy(self_v, self_out.at[pl.ds(base, K)])
        pltpu.sync_copy(sum_v, sum_out.at[pl.ds(base, K)])


def _sc_gather(nodes, neigh_flat, features):
    f = pl.kernel(
        _sc_body,
        out_type=(jax.ShapeDtypeStruct((B, D), jnp.float32),
                  jax.ShapeDtypeStruct((B, D), jnp.float32)),
        mesh=plsc.VectorSubcoreMesh(core_axis_name="c", subcore_axis_name="s",
                                    num_cores=NC, num_subcores=NS),
        scratch_types=(
            pltpu.VMEM((K,), jnp.int32),
            pltpu.VMEM((GPC, GLEN), jnp.int32),
            pltpu.VMEM((K * S, D), jnp.float32),
            pltpu.VMEM((K, D), jnp.float32),
            pltpu.VMEM((K, D), jnp.float32),
            pltpu.SemaphoreType.DMA,
        ),
    )
    return f(nodes, neigh_flat, features)


BM = 1024


def _tc_body(xs_ref, xm_ref, ws_ref, wn_ref, wc_ref, o_ref):
    h = jnp.dot(xs_ref[...], ws_ref[...], preferred_element_type=jnp.float32)
    h = h + jnp.dot(xm_ref[...], wn_ref[...], preferred_element_type=jnp.float32)
    h = jnp.maximum(h, 0.0)
    o_ref[...] = jnp.dot(h, wc_ref[...], preferred_element_type=jnp.float32)


def _tc_head(xs, xm, ws_t, wn_t, wc_t):
    return pl.pallas_call(
        _tc_body,
        grid=(B // BM,),
        in_specs=[
            pl.BlockSpec((BM, D), lambda i: (i, 0)),
            pl.BlockSpec((BM, D), lambda i: (i, 0)),
            pl.BlockSpec((D, D), lambda i: (0, 0)),
            pl.BlockSpec((D, D), lambda i: (0, 0)),
            pl.BlockSpec((D, C), lambda i: (0, 0)),
        ],
        out_specs=pl.BlockSpec((BM, C), lambda i: (i, 0)),
        out_shape=jax.ShapeDtypeStruct((B, C), jnp.float32),
    )(xs, xm, ws_t, wn_t, wc_t)


def kernel(nodes, neigh_idx, features, W_enc, weight):
    neigh2 = neigh_idx.reshape(-1, GLEN)
    self_out, sum_out = _sc_gather(nodes, neigh2, features)
    ws_t = W_enc[:, :D].T
    wn_t = W_enc[:, D:].T * jnp.float32(1.0 / S)
    wc_t = weight.T
    return _tc_head(self_out, sum_out, ws_t, wn_t, wc_t)


# speedup vs baseline: 11.1740x; 1.5584x over previous
import jax
import jax.numpy as jnp
from jax import lax
from jax.experimental import pallas as pl
from jax.experimental.pallas import tpu as pltpu
from jax.experimental.pallas import tpu_sc as plsc

B = 16384
D = 128
S = 25
C = 64
NC = 2
NS = 16
NW = NC * NS
PER_W = B // NW
K = 16
CHUNKS = PER_W // K
ROWS = K * S + K
GPC = 4
GLEN = ROWS // GPC
NLANE = 16
NVD = D // NLANE


def _sc_body(idx_hbm, feat_hbm, self_out, sum_out,
             idx0, idx1, rows0, rows1, sum_v, sem0, sem1):
    cid = lax.axis_index("c")
    sid = lax.axis_index("s")
    wid = sid * NC + cid
    idxs = (idx0, idx1)
    rows = (rows0, rows1)
    sems = (sem0, sem1)

    def fire(c, b):
        t = wid * CHUNKS + c
        pltpu.sync_copy(idx_hbm.at[pl.ds(t * GPC, GPC)], idxs[b])
        for j in range(GPC):
            pltpu.async_copy(feat_hbm.at[idxs[b].at[j]],
                             rows[b].at[pl.ds(j * GLEN, GLEN)], sems[b])

    def drain(b):
        for j in range(GPC):
            pltpu.make_async_copy(feat_hbm.at[pl.ds(0, GLEN)],
                                  rows[b].at[pl.ds(j * GLEN, GLEN)],
                                  sems[b]).wait()

    def compute(c, b):
        @pl.loop(0, K)
        def _elem(k):
            r0 = k * S
            acc = tuple(rows[b][r0, pl.ds(NLANE * d, NLANE)]
                        for d in range(NVD))

            def _sbody(s2, a):
                return tuple(a[d] + rows[b][r0 + s2, pl.ds(NLANE * d, NLANE)]
                             for d in range(NVD))

            acc = lax.fori_loop(1, S, _sbody, acc)
            for d in range(NVD):
                sum_v[k, pl.ds(NLANE * d, NLANE)] = acc[d]

        base = (wid * CHUNKS + c) * K
        pltpu.sync_copy(rows[b].at[pl.ds(K * S, K)], self_out.at[pl.ds(base, K)])
        pltpu.sync_copy(sum_v, sum_out.at[pl.ds(base, K)])

    fire(0, 0)

    @pl.loop(0, CHUNKS, step=2)
    def _outer(cb):
        for b in range(2):
            c = cb + b

            @pl.when(c + 1 < CHUNKS)
            def _():
                fire(c + 1, 1 - b)

            drain(b)
            compute(c, b)


def _sc_gather(idx_packed, features):
    f = pl.kernel(
        _sc_body,
        out_type=(jax.ShapeDtypeStruct((B, D), jnp.float32),
                  jax.ShapeDtypeStruct((B, D), jnp.float32)),
        mesh=plsc.VectorSubcoreMesh(core_axis_name="c", subcore_axis_name="s",
                                    num_cores=NC, num_subcores=NS),
        scratch_types=(
            pltpu.VMEM((GPC, GLEN), jnp.int32),
            pltpu.VMEM((GPC, GLEN), jnp.int32),
            pltpu.VMEM((ROWS, D), jnp.float32),
            pltpu.VMEM((ROWS, D), jnp.float32),
            pltpu.VMEM((K, D), jnp.float32),
            pltpu.SemaphoreType.DMA,
            pltpu.SemaphoreType.DMA,
        ),
    )
    return f(idx_packed, features)


BM = 1024


def _tc_body(xs_ref, xm_ref, ws_ref, wn_ref, wc_ref, o_ref):
    h = jnp.dot(xs_ref[...], ws_ref[...], preferred_element_type=jnp.float32)
    h = h + jnp.dot(xm_ref[...], wn_ref[...], preferred_element_type=jnp.float32)
    h = jnp.maximum(h, 0.0)
    o_ref[...] = jnp.dot(h, wc_ref[...], preferred_element_type=jnp.float32)


def _tc_head(xs, xm, ws_t, wn_t, wc_t):
    return pl.pallas_call(
        _tc_body,
        grid=(B // BM,),
        in_specs=[
            pl.BlockSpec((BM, D), lambda i: (i, 0)),
            pl.BlockSpec((BM, D), lambda i: (i, 0)),
            pl.BlockSpec((D, D), lambda i: (0, 0)),
            pl.BlockSpec((D, D), lambda i: (0, 0)),
            pl.BlockSpec((D, C), lambda i: (0, 0)),
        ],
        out_specs=pl.BlockSpec((BM, C), lambda i: (i, 0)),
        out_shape=jax.ShapeDtypeStruct((B, C), jnp.float32),
    )(xs, xm, ws_t, wn_t, wc_t)


def kernel(nodes, neigh_idx, features, W_enc, weight):
    idx_packed = jnp.concatenate(
        [neigh_idx.reshape(B // K, K * S), nodes.reshape(B // K, K)], axis=1
    ).reshape(-1, GLEN)
    self_out, sum_out = _sc_gather(idx_packed, features)
    ws_t = W_enc[:, :D].T
    wn_t = W_enc[:, D:].T * jnp.float32(1.0 / S)
    wc_t = weight.T
    return _tc_head(self_out, sum_out, ws_t, wn_t, wc_t)


# speedup vs baseline: 11.2874x; 1.0101x over previous
import jax
import jax.numpy as jnp
from jax import lax
from jax.experimental import pallas as pl
from jax.experimental.pallas import tpu as pltpu
from jax.experimental.pallas import tpu_sc as plsc

B = 16384
D = 128
S = 25
C = 64
NC = 2
NS = 16
NW = NC * NS
PER_W = B // NW
K = 16
CHUNKS = PER_W // K
ROWS = K * S + K
GPC = 4
GLEN = ROWS // GPC
NLANE = 16
NVD = D // NLANE


def _sc_body(idx_hbm, feat_hbm, self_out, sum_out,
             idx0, idx1, rows0, rows1, sum0, sum1,
             sem0, sem1, osem0, osem1):
    cid = lax.axis_index("c")
    sid = lax.axis_index("s")
    wid = sid * NC + cid
    idxs = (idx0, idx1)
    rows = (rows0, rows1)
    sums = (sum0, sum1)
    sems = (sem0, sem1)
    osems = (osem0, osem1)

    def fire(c, b):
        t = wid * CHUNKS + c
        pltpu.sync_copy(idx_hbm.at[pl.ds(t * GPC, GPC)], idxs[b])
        for j in range(GPC):
            pltpu.async_copy(feat_hbm.at[idxs[b].at[j]],
                             rows[b].at[pl.ds(j * GLEN, GLEN)], sems[b])

    def drain(b):
        for j in range(GPC):
            pltpu.make_async_copy(feat_hbm.at[pl.ds(0, GLEN)],
                                  rows[b].at[pl.ds(j * GLEN, GLEN)],
                                  sems[b]).wait()

    def compute(c, b):
        rb = rows[b]
        sb = sums[b]

        @plsc.parallel_loop(0, K, unroll=2)
        def _elem(k):
            r0 = k * S
            acc = tuple(rb[r0, pl.ds(NLANE * d, NLANE)] for d in range(NVD))

            def _sbody(s2, a):
                return tuple(a[d] + rb[r0 + s2, pl.ds(NLANE * d, NLANE)]
                             for d in range(NVD))

            acc = lax.fori_loop(1, S, _sbody, acc, unroll=8)
            for d in range(NVD):
                sb[k, pl.ds(NLANE * d, NLANE)] = acc[d]

        base = (wid * CHUNKS + c) * K
        pltpu.async_copy(rb.at[pl.ds(K * S, K)], self_out.at[pl.ds(base, K)],
                         osems[b])
        pltpu.async_copy(sb, sum_out.at[pl.ds(base, K)], osems[b])

    def drain_out(b):
        pltpu.make_async_copy(feat_hbm.at[pl.ds(0, K)], sums[b],
                              osems[b]).wait()
        pltpu.make_async_copy(feat_hbm.at[pl.ds(0, K)],
                              rows[b].at[pl.ds(K * S, K)], osems[b]).wait()

    fire(0, 0)

    @pl.loop(0, CHUNKS, step=2)
    def _outer(cb):
        for b in range(2):
            c = cb + b

            @pl.when(c > 0)
            def _():
                drain_out(1 - b)

            @pl.when(c + 1 < CHUNKS)
            def _():
                fire(c + 1, 1 - b)

            drain(b)
            compute(c, b)

    drain_out(1)


def _sc_gather(idx_packed, features):
    f = pl.kernel(
        _sc_body,
        out_type=(jax.ShapeDtypeStruct((B, D), jnp.float32),
                  jax.ShapeDtypeStruct((B, D), jnp.float32)),
        mesh=plsc.VectorSubcoreMesh(core_axis_name="c", subcore_axis_name="s",
                                    num_cores=NC, num_subcores=NS),
        scratch_types=(
            pltpu.VMEM((GPC, GLEN), jnp.int32),
            pltpu.VMEM((GPC, GLEN), jnp.int32),
            pltpu.VMEM((ROWS, D), jnp.float32),
            pltpu.VMEM((ROWS, D), jnp.float32),
            pltpu.VMEM((K, D), jnp.float32),
            pltpu.VMEM((K, D), jnp.float32),
            pltpu.SemaphoreType.DMA,
            pltpu.SemaphoreType.DMA,
            pltpu.SemaphoreType.DMA,
            pltpu.SemaphoreType.DMA,
        ),
    )
    return f(idx_packed, features)


BM = 1024


def _tc_body(xs_ref, xm_ref, ws_ref, wn_ref, wc_ref, o_ref):
    h = jnp.dot(xs_ref[...], ws_ref[...], preferred_element_type=jnp.float32)
    h = h + jnp.dot(xm_ref[...], wn_ref[...], preferred_element_type=jnp.float32)
    h = jnp.maximum(h, 0.0)
    o_ref[...] = jnp.dot(h, wc_ref[...], preferred_element_type=jnp.float32)


def _tc_head(xs, xm, ws_t, wn_t, wc_t):
    return pl.pallas_call(
        _tc_body,
        grid=(B // BM,),
        in_specs=[
            pl.BlockSpec((BM, D), lambda i: (i, 0)),
            pl.BlockSpec((BM, D), lambda i: (i, 0)),
            pl.BlockSpec((D, D), lambda i: (0, 0)),
            pl.BlockSpec((D, D), lambda i: (0, 0)),
            pl.BlockSpec((D, C), lambda i: (0, 0)),
        ],
        out_specs=pl.BlockSpec((BM, C), lambda i: (i, 0)),
        out_shape=jax.ShapeDtypeStruct((B, C), jnp.float32),
    )(xs, xm, ws_t, wn_t, wc_t)


def kernel(nodes, neigh_idx, features, W_enc, weight):
    idx_packed = jnp.concatenate(
        [neigh_idx.reshape(B // K, K * S), nodes.reshape(B // K, K)], axis=1
    ).reshape(-1, GLEN)
    self_out, sum_out = _sc_gather(idx_packed, features)
    ws_t = W_enc[:, :D].T
    wn_t = W_enc[:, D:].T * jnp.float32(1.0 / S)
    wc_t = weight.T
    return _tc_head(self_out, sum_out, ws_t, wn_t, wc_t)
